# baseline (speedup 1.0000x reference)
import functools

import jax
import jax.numpy as jnp
from jax import lax
from jax.experimental import pallas as pl
from jax.experimental.pallas import tpu as pltpu
from jax.experimental.pallas import tpu_sc as plsc

_NC, _NS, _L = 2, 16, 16
_NW = _NC * _NS
_T = 8


def _sc_add(emb_flat, pos_flat, B, S, E):
    s_per_w = S // _NW
    n_t = s_per_w // _T
    chunk = _T * E
    mesh = plsc.VectorSubcoreMesh(core_axis_name="c", subcore_axis_name="s")

    @functools.partial(
        pl.kernel,
        mesh=mesh,
        out_type=jax.ShapeDtypeStruct((B * S * E,), jnp.float32),
        scratch_types=[
            pltpu.VMEM((chunk,), jnp.float32),
            pltpu.VMEM((chunk,), jnp.float32),
        ],
    )
    def run(emb_hbm, pos_hbm, out_hbm, pos_v, emb_v):
        w = lax.axis_index("s") * _NC + lax.axis_index("c")

        def t_loop(t, carry):
            s0 = (w * s_per_w + t * _T) * E
            pltpu.sync_copy(pos_hbm.at[pl.ds(s0, chunk)], pos_v)

            def b_loop(b, carry):
                row0 = b * S * E + s0
                pltpu.sync_copy(emb_hbm.at[pl.ds(row0, chunk)], emb_v)

                @plsc.parallel_loop(0, chunk // _L, unroll=8)
                def _(i):
                    sl = pl.ds(i * _L, _L)
                    emb_v[sl] = emb_v[sl] + pos_v[sl]

                pltpu.sync_copy(emb_v, out_hbm.at[pl.ds(row0, chunk)])
                return carry

            return lax.fori_loop(0, B, b_loop, carry)

        lax.fori_loop(0, n_t, t_loop, 0)

    return run(emb_flat, pos_flat)


_TS = 512


def _tc_body(emb_ref, pos_ref, out_ref):
    out_ref[0] = emb_ref[0] + pos_ref[...]


def _tc_add(embeddings, pos_table, B, S, E):
    grid = (S // _TS, B)
    return pl.pallas_call(
        _tc_body,
        grid=grid,
        in_specs=[
            pl.BlockSpec((1, _TS, E), lambda s, b: (b, s, 0)),
            pl.BlockSpec((_TS, E), lambda s, b: (s, 0)),
        ],
        out_specs=pl.BlockSpec((1, _TS, E), lambda s, b: (b, s, 0)),
        out_shape=jax.ShapeDtypeStruct((B, S, E), embeddings.dtype),
    )(embeddings, pos_table)


def kernel(embeddings, pos_table):
    B, S, E = embeddings.shape
    emb_flat = embeddings.reshape(B * S * E)
    pos_flat = pos_table[:S].reshape(S * E)
    out = _sc_add(emb_flat, pos_flat, B, S, E)
    return out.reshape(B, S, E)


# speedup vs baseline: 1.2737x; 1.2737x over previous
import functools

import jax
import jax.numpy as jnp
from jax import lax
from jax.experimental import pallas as pl
from jax.experimental.pallas import tpu as pltpu
from jax.experimental.pallas import tpu_sc as plsc

_NC, _NS, _L = 2, 16, 16
_NW = _NC * _NS
_T = 8


def _sc_add(emb_flat, pos_flat, B, S, E):
    s_per_w = S // _NW
    n_t = s_per_w // _T
    n_steps = n_t * B
    chunk = _T * E
    mesh = plsc.VectorSubcoreMesh(core_axis_name="c", subcore_axis_name="s")

    @functools.partial(
        pl.kernel,
        mesh=mesh,
        out_type=jax.ShapeDtypeStruct((B * S * E,), jnp.float32),
        scratch_types=[
            [pltpu.VMEM((chunk,), jnp.float32) for _ in range(4)],
            [pltpu.VMEM((chunk,), jnp.float32) for _ in range(2)],
            [pltpu.SemaphoreType.DMA for _ in range(4)],
            [pltpu.SemaphoreType.DMA for _ in range(4)],
            [pltpu.SemaphoreType.DMA for _ in range(2)],
        ],
    )
    def run(emb_hbm, pos_hbm, out_hbm, bufs, pbufs, lsems, ssems, psems):
        w = lax.axis_index("s") * _NC + lax.axis_index("c")
        base_s = w * s_per_w

        def emb_sl(step):
            t, b = divmod(step, B)
            return pl.ds((b * S + base_s + t * _T) * E, chunk)

        pos_d = [
            pltpu.async_copy(pos_hbm.at[pl.ds((base_s + t * _T) * E, chunk)],
                             pbufs[t % 2], psems[t % 2])
            for t in range(2)
        ]
        load_d = [None] * n_steps
        store_d = [None] * n_steps
        for s in range(2):
            load_d[s] = pltpu.async_copy(emb_hbm.at[emb_sl(s)],
                                         bufs[s % 4], lsems[s % 4])
        for s in range(n_steps):
            t, b = divmod(s, B)
            if b == 0:
                pos_d[t].wait()
            load_d[s].wait()
            buf, pbuf = bufs[s % 4], pbufs[t % 2]

            @plsc.parallel_loop(0, chunk // _L, unroll=8)
            def _(i):
                sl = pl.ds(i * _L, _L)
                buf[sl] = buf[sl] + pbuf[sl]

            store_d[s] = pltpu.async_copy(buf, out_hbm.at[emb_sl(s)],
                                          ssems[s % 4])
            if b == B - 1 and t + 2 < n_t:
                pos_d.append(
                    pltpu.async_copy(
                        pos_hbm.at[pl.ds((base_s + (t + 2) * _T) * E, chunk)],
                        pbufs[t % 2], psems[t % 2]))
            ns = s + 2
            if ns < n_steps:
                if ns >= 4:
                    store_d[ns - 4].wait()
                load_d[ns] = pltpu.async_copy(emb_hbm.at[emb_sl(ns)],
                                              bufs[ns % 4], lsems[ns % 4])
        for s in range(n_steps - 4, n_steps):
            store_d[s].wait()

    return run(emb_flat, pos_flat)


_TS = 512


def _tc_body(emb_ref, pos_ref, out_ref):
    out_ref[0] = emb_ref[0] + pos_ref[...]


def _tc_add(embeddings, pos_table, B, S, E):
    grid = (S // _TS, B)
    return pl.pallas_call(
        _tc_body,
        grid=grid,
        in_specs=[
            pl.BlockSpec((1, _TS, E), lambda s, b: (b, s, 0)),
            pl.BlockSpec((_TS, E), lambda s, b: (s, 0)),
        ],
        out_specs=pl.BlockSpec((1, _TS, E), lambda s, b: (b, s, 0)),
        out_shape=jax.ShapeDtypeStruct((B, S, E), embeddings.dtype),
    )(embeddings, pos_table)


def kernel(embeddings, pos_table):
    B, S, E = embeddings.shape
    emb_flat = embeddings.reshape(B * S * E)
    pos_flat = pos_table[:S].reshape(S * E)
    out = _sc_add(emb_flat, pos_flat, B, S, E)
    return out.reshape(B, S, E)


# speedup vs baseline: 3.4246x; 2.6886x over previous
import functools

import jax
import jax.numpy as jnp
from jax import lax
from jax.experimental import pallas as pl
from jax.experimental.pallas import tpu as pltpu
from jax.experimental.pallas import tpu_sc as plsc

_NC, _NS, _L = 2, 16, 16
_NW = _NC * _NS
_T = 8


def _sc_add(emb2d, pos2d, B, S, E):
    s_per_w = S // _NW
    n_t = s_per_w // _T
    n_steps = n_t * B
    n_vec = _T * E // _L
    mesh = plsc.VectorSubcoreMesh(core_axis_name="c", subcore_axis_name="s")

    @functools.partial(
        pl.kernel,
        mesh=mesh,
        out_type=jax.ShapeDtypeStruct((B * S, E), jnp.float32),
        compiler_params=pltpu.CompilerParams(use_tc_tiling_on_sc=True),
        scratch_types=[
            [pltpu.VMEM((_T, E), jnp.float32) for _ in range(4)],
            [pltpu.VMEM((_T, E), jnp.float32) for _ in range(2)],
            [pltpu.SemaphoreType.DMA for _ in range(4)],
            [pltpu.SemaphoreType.DMA for _ in range(4)],
            [pltpu.SemaphoreType.DMA for _ in range(2)],
        ],
    )
    def run(emb_hbm, pos_hbm, out_hbm, bufs, pbufs, lsems, ssems, psems):
        w = lax.axis_index("s") * _NC + lax.axis_index("c")
        base_s = w * s_per_w

        def emb_sl(step):
            t, b = divmod(step, B)
            return pl.ds(b * S + base_s + t * _T, _T)

        pos_d = [
            pltpu.async_copy(pos_hbm.at[pl.ds(base_s + t * _T, _T)],
                             pbufs[t % 2], psems[t % 2])
            for t in range(2)
        ]
        load_d = [None] * n_steps
        store_d = [None] * n_steps
        for s in range(2):
            load_d[s] = pltpu.async_copy(emb_hbm.at[emb_sl(s)],
                                         bufs[s % 4], lsems[s % 4])
        for s in range(n_steps):
            t, b = divmod(s, B)
            if b == 0:
                pos_d[t].wait()
            load_d[s].wait()
            buf, pbuf = bufs[s % 4], pbufs[t % 2]

            @plsc.parallel_loop(0, n_vec, unroll=8)
            def _(i):
                r = i // (E // _L)
                sl = pl.ds((i % (E // _L)) * _L, _L)
                buf[r, sl] = buf[r, sl] + pbuf[r, sl]

            store_d[s] = pltpu.async_copy(buf, out_hbm.at[emb_sl(s)],
                                          ssems[s % 4])
            if b == B - 1 and t + 2 < n_t:
                pos_d.append(
                    pltpu.async_copy(pos_hbm.at[pl.ds(base_s + (t + 2) * _T, _T)],
                                     pbufs[t % 2], psems[t % 2]))
            ns = s + 2
            if ns < n_steps:
                if ns >= 4:
                    store_d[ns - 4].wait()
                load_d[ns] = pltpu.async_copy(emb_hbm.at[emb_sl(ns)],
                                              bufs[ns % 4], lsems[ns % 4])
        for s in range(n_steps - 4, n_steps):
            store_d[s].wait()

    return run(emb2d, pos2d)


_TS = 512


def _tc_body(emb_ref, pos_ref, out_ref):
    out_ref[0] = emb_ref[0] + pos_ref[...]


def _tc_add(embeddings, pos_table, B, S, E):
    grid = (S // _TS, B)
    return pl.pallas_call(
        _tc_body,
        grid=grid,
        in_specs=[
            pl.BlockSpec((1, _TS, E), lambda s, b: (b, s, 0)),
            pl.BlockSpec((_TS, E), lambda s, b: (s, 0)),
        ],
        out_specs=pl.BlockSpec((1, _TS, E), lambda s, b: (b, s, 0)),
        out_shape=jax.ShapeDtypeStruct((B, S, E), embeddings.dtype),
    )(embeddings, pos_table)


def kernel(embeddings, pos_table):
    B, S, E = embeddings.shape
    emb2d = embeddings.reshape(B * S, E)
    pos2d = pos_table[:S]
    out = _sc_add(emb2d, pos2d, B, S, E)
    return out.reshape(B, S, E)
